# Initial kernel scaffold; baseline (speedup 1.0000x reference)
import jax
import jax.numpy as jnp
from jax.experimental import pallas as pl


def kernel(sentence_batch, emb_table):
    raise NotImplementedError("write your pallas kernel here")


# speedup vs baseline: 1.1910x; 1.1910x over previous
import jax
import jax.numpy as jnp
from jax import lax
from jax.experimental import pallas as pl
from jax.experimental.pallas import tpu as pltpu
from jax.experimental.pallas import tpu_sc as plsc

N = 128
L = 200
PATCH = 16
IMG = 384
EMB_DIM = 768
ROWS = 24
LANES = 16
VECS = EMB_DIM // LANES

NUM_CORES = 2
NUM_SUBCORES = 16
NUM_WORKERS = NUM_CORES * NUM_SUBCORES
SAMPLES_PER_WORKER = N // NUM_WORKERS


def _tanh16(x):
  return 1.0 - 2.0 / (jnp.exp(2.0 * x) + 1.0)


def _sc_body(tok_hbm, table_hbm, out_hbm, idx24, rows, band, sem):
  wid = lax.axis_index("s") * NUM_CORES + lax.axis_index("c")

  def do_sample(i, _):
    n = wid * SAMPLES_PER_WORKER + i

    def do_band(r, _):
      off = pl.multiple_of(n * L + r * ROWS, 8)
      pltpu.sync_copy(tok_hbm.at[pl.ds(off, ROWS)], idx24)
      pltpu.async_copy(table_hbm.at[idx24], rows, sem).wait()

      def do_col(c, _):
        cbase = pl.multiple_of(c * PATCH, 16)
        for j in range(VECS):
          ch, y = j // PATCH, j % PATCH
          band[ch, y, pl.ds(cbase, LANES)] = _tanh16(
              rows[c, pl.ds(j * LANES, LANES)])
        return 0

      lax.fori_loop(0, ROWS, do_col, 0)
      rbase = pl.multiple_of(r * PATCH, 16)
      pltpu.sync_copy(band, out_hbm.at[n, :, pl.ds(rbase, PATCH), :])
      return 0

    lax.fori_loop(0, 8, do_band, 0)

    off8 = pl.multiple_of(n * L + 192, 8)
    pltpu.sync_copy(tok_hbm.at[pl.ds(off8, 8)], idx24.at[pl.ds(0, 8)])
    pltpu.async_copy(table_hbm.at[idx24.at[pl.ds(0, 8)]],
                     rows.at[pl.ds(0, 8)], sem).wait()

    def do_col8(c, _):
      cbase = pl.multiple_of(c * PATCH, 16)
      for j in range(VECS):
        ch, y = j // PATCH, j % PATCH
        band[ch, y, pl.ds(cbase, LANES)] = _tanh16(
            rows[c, pl.ds(j * LANES, LANES)])
      return 0

    lax.fori_loop(0, 7, do_col8, 0)

    for j in range(VECS):
      ch, y = j // PATCH, j % PATCH
      band[ch, y, pl.ds(7 * PATCH, LANES)] = _tanh16(
          rows[7, pl.ds(j * LANES, LANES)])

    def fan_col(c, _):
      cbase = pl.multiple_of(c * PATCH, 16)
      for j in range(VECS):
        ch, y = j // PATCH, j % PATCH
        band[ch, y, pl.ds(cbase, LANES)] = band[ch, y, pl.ds(7 * PATCH, LANES)]
      return 0

    lax.fori_loop(8, ROWS, fan_col, 0)
    pltpu.sync_copy(band, out_hbm.at[n, :, pl.ds(8 * PATCH, PATCH), :])

    lax.fori_loop(0, 7, fan_col, 0)

    def write_band(r, _):
      rbase = pl.multiple_of(r * PATCH, 16)
      pltpu.sync_copy(band, out_hbm.at[n, :, pl.ds(rbase, PATCH), :])
      return 0

    lax.fori_loop(9, ROWS, write_band, 0)
    return 0

  lax.fori_loop(0, SAMPLES_PER_WORKER, do_sample, 0)


@jax.jit
def kernel(sentence_batch, emb_table):
  mesh = plsc.VectorSubcoreMesh(core_axis_name="c", subcore_axis_name="s",
                                num_cores=NUM_CORES,
                                num_subcores=NUM_SUBCORES)
  run = pl.kernel(
      _sc_body,
      out_type=jax.ShapeDtypeStruct((N, 3, IMG, IMG), jnp.float32),
      mesh=mesh,
      scratch_types=[
          pltpu.VMEM((ROWS,), jnp.int32),
          pltpu.VMEM((ROWS, EMB_DIM), jnp.float32),
          pltpu.VMEM((3, PATCH, IMG), jnp.float32),
          pltpu.SemaphoreType.DMA,
      ],
  )
  return run(sentence_batch.astype(jnp.int32).reshape(N * L), emb_table)


# speedup vs baseline: 1.3365x; 1.1221x over previous
import jax
import jax.numpy as jnp
from jax import lax
from jax.experimental import pallas as pl
from jax.experimental.pallas import tpu as pltpu
from jax.experimental.pallas import tpu_sc as plsc

N = 128
L = 200
PATCH = 16
IMG = 384
EMB_DIM = 768
ROWS = 24
LANES = 16
VECS = EMB_DIM // LANES

NUM_CORES = 2
NUM_SUBCORES = 16
NUM_WORKERS = NUM_CORES * NUM_SUBCORES
SPW = N // NUM_WORKERS


def _tanh16(x):
  return 1.0 - 2.0 / (jnp.exp(2.0 * x) + 1.0)


def _sc_body(tok_hbm, table_hbm, out_hbm,
             tok_v, rows0, rows1, rows8, band_a0, band_a1, band_b,
             gs0, gs1, gs8, ws0, ws1, rsem):
  wid = lax.axis_index("s") * NUM_CORES + lax.axis_index("c")
  rows_bufs = (rows0, rows1)
  gsems = (gs0, gs1)
  band_bufs = (band_a0, band_a1)
  wsems = (ws0, ws1)

  tok_off = pl.multiple_of(wid * (SPW * L), 8)
  pltpu.sync_copy(tok_hbm.at[pl.ds(tok_off, SPW * L)], tok_v)

  def idx(i, start, count):
    return tok_v.at[pl.ds(pl.multiple_of(i * L + start, 8), count)]

  def gather(i, r, buf, sem):
    if r < 8:
      return pltpu.make_async_copy(table_hbm.at[idx(i, r * ROWS, ROWS)],
                                   buf, sem)
    return pltpu.make_async_copy(table_hbm.at[idx(i, 192, 8)], buf, sem)

  def band_write(n, r, buf, sem):
    return pltpu.make_async_copy(
        buf, out_hbm.at[n, :, pl.ds(pl.multiple_of(r * PATCH, 16), PATCH), :],
        sem)

  def build_cols(rows, band, lo, hi):
    def do_col(c, _):
      cbase = pl.multiple_of(c * PATCH, 16)
      for j in range(VECS):
        ch, y = j // PATCH, j % PATCH
        band[ch, y, pl.ds(cbase, LANES)] = _tanh16(
            rows[c, pl.ds(j * LANES, LANES)])
      return 0
    lax.fori_loop(lo, hi, do_col, 0)

  def do_sample(i, _):
    n = wid * SPW + i
    gather(i, 0, rows0, gs0).start()

    for r in range(8):
      if r < 7:
        gather(i, r + 1, rows_bufs[(r + 1) % 2], gsems[(r + 1) % 2]).start()
      else:
        gather(i, 8, rows8, gs8).start()
      gather(i, r, rows_bufs[r % 2], gsems[r % 2]).wait()

      if r >= 2:
        band_write(n, r, band_bufs[r % 2], wsems[r % 2]).wait()
      else:
        @pl.when(i > 0)
        def _():
          band_write(n, r, band_bufs[r % 2], wsems[r % 2]).wait()

      build_cols(rows_bufs[r % 2], band_bufs[r % 2], 0, ROWS)
      band_write(n, r, band_bufs[r % 2], wsems[r % 2]).start()

    gather(i, 8, rows8, gs8).wait()

    @pl.when(i > 0)
    def _():
      for _k in range(ROWS - 9):
        band_write(n, 9, band_b, rsem).wait()

    for j in range(VECS):
      ch, y = j // PATCH, j % PATCH
      band_b[ch, y, pl.ds(0, LANES)] = _tanh16(rows8[7, pl.ds(j * LANES, LANES)])

    def fan_col(c, _):
      cbase = pl.multiple_of(c * PATCH, 16)
      for j in range(VECS):
        ch, y = j // PATCH, j % PATCH
        band_b[ch, y, pl.ds(cbase, LANES)] = band_b[ch, y, pl.ds(0, LANES)]
      return 0
    lax.fori_loop(1, ROWS, fan_col, 0)

    def write_rep(r, _):
      band_write(n, r, band_b, rsem).start()
      return 0
    lax.fori_loop(9, ROWS, write_rep, 0)

    band_write(n, 8, band_a0, ws0).wait()
    build_cols(rows8, band_a0, 0, 7)

    def fill_199(c, _):
      cbase = pl.multiple_of(c * PATCH, 16)
      for j in range(VECS):
        ch, y = j // PATCH, j % PATCH
        band_a0[ch, y, pl.ds(cbase, LANES)] = band_b[ch, y, pl.ds(0, LANES)]
      return 0
    lax.fori_loop(7, ROWS, fill_199, 0)
    band_write(n, 8, band_a0, ws0).start()
    return 0

  lax.fori_loop(0, SPW, do_sample, 0)

  last = wid * SPW + (SPW - 1)
  band_write(last, 8, band_a0, ws0).wait()
  band_write(last, 7, band_a1, ws1).wait()
  for _k in range(ROWS - 9):
    band_write(last, 9, band_b, rsem).wait()


@jax.jit
def kernel(sentence_batch, emb_table):
  mesh = plsc.VectorSubcoreMesh(core_axis_name="c", subcore_axis_name="s",
                                num_cores=NUM_CORES,
                                num_subcores=NUM_SUBCORES)
  run = pl.kernel(
      _sc_body,
      out_type=jax.ShapeDtypeStruct((N, 3, IMG, IMG), jnp.float32),
      mesh=mesh,
      scratch_types=[
          pltpu.VMEM((SPW * L,), jnp.int32),
          pltpu.VMEM((ROWS, EMB_DIM), jnp.float32),
          pltpu.VMEM((ROWS, EMB_DIM), jnp.float32),
          pltpu.VMEM((8, EMB_DIM), jnp.float32),
          pltpu.VMEM((3, PATCH, IMG), jnp.float32),
          pltpu.VMEM((3, PATCH, IMG), jnp.float32),
          pltpu.VMEM((3, PATCH, IMG), jnp.float32),
          pltpu.SemaphoreType.DMA,
          pltpu.SemaphoreType.DMA,
          pltpu.SemaphoreType.DMA,
          pltpu.SemaphoreType.DMA,
          pltpu.SemaphoreType.DMA,
          pltpu.SemaphoreType.DMA,
      ],
  )
  return run(sentence_batch.astype(jnp.int32).reshape(N * L), emb_table)


# speedup vs baseline: 6.4645x; 4.8369x over previous
import jax
import jax.numpy as jnp
from jax import lax
from jax.experimental import pallas as pl
from jax.experimental.pallas import tpu as pltpu
from jax.experimental.pallas import tpu_sc as plsc

N = 128
L = 200
PATCH = 16
IMG = 384
EMB_DIM = 768
ROWS = 24
LANES = 16
VECS = EMB_DIM // LANES

NUM_CORES = 2
NUM_SUBCORES = 16
NUM_WORKERS = NUM_CORES * NUM_SUBCORES
SPW = N // NUM_WORKERS


def _tanh16(x):
  return 1.0 - 2.0 / (jnp.exp(2.0 * x) + 1.0)


def _sc_body(tok_hbm, table_hbm, out_hbm,
             tok_v, rows0, rows1, rows8, band_a0, band_a1, band_b,
             gs0, gs1, gs8, ws0, ws1, rsem):
  wid = lax.axis_index("s") * NUM_CORES + lax.axis_index("c")
  rows_bufs = (rows0, rows1)
  gsems = (gs0, gs1)
  band_bufs = (band_a0, band_a1)
  wsems = (ws0, ws1)

  tok_off = pl.multiple_of(wid * (SPW * L), 8)
  pltpu.sync_copy(tok_hbm.at[pl.ds(tok_off, SPW * L)], tok_v)

  def idx(i, start, count):
    return tok_v.at[pl.ds(pl.multiple_of(i * L + start, 8), count)]

  def gather(i, r, buf, sem):
    if r < 8:
      return pltpu.make_async_copy(table_hbm.at[idx(i, r * ROWS, ROWS)],
                                   buf, sem)
    return pltpu.make_async_copy(table_hbm.at[idx(i, 192, 8)], buf, sem)

  def band_write(n, r, buf, sem):
    return pltpu.make_async_copy(
        buf, out_hbm.at[n, :, pl.ds(pl.multiple_of(r * PATCH, 16), PATCH), :],
        sem)

  def build_cols(rows, band, lo, hi):
    def do_col(c, _):
      cbase = pl.multiple_of(c * PATCH, 16)

      @plsc.parallel_loop(0, VECS, unroll=4)
      def _(j):
        ch = j // PATCH
        y = j - ch * PATCH
        band[ch, y, pl.ds(cbase, LANES)] = _tanh16(
            rows[c, pl.ds(pl.multiple_of(j * LANES, 16), LANES)])

      return 0
    lax.fori_loop(lo, hi, do_col, 0)

  def do_sample(i, _):
    n = wid * SPW + i
    gather(i, 0, rows0, gs0).start()

    for r in range(8):
      if r < 7:
        gather(i, r + 1, rows_bufs[(r + 1) % 2], gsems[(r + 1) % 2]).start()
      else:
        gather(i, 8, rows8, gs8).start()
      gather(i, r, rows_bufs[r % 2], gsems[r % 2]).wait()

      if r >= 2:
        band_write(n, r, band_bufs[r % 2], wsems[r % 2]).wait()
      else:
        @pl.when(i > 0)
        def _():
          band_write(n, r, band_bufs[r % 2], wsems[r % 2]).wait()

      build_cols(rows_bufs[r % 2], band_bufs[r % 2], 0, ROWS)
      band_write(n, r, band_bufs[r % 2], wsems[r % 2]).start()

    gather(i, 8, rows8, gs8).wait()

    @pl.when(i > 0)
    def _():
      for _k in range(ROWS - 9):
        band_write(n, 9, band_b, rsem).wait()

    @plsc.parallel_loop(0, VECS, unroll=4)
    def _(j):
      ch = j // PATCH
      y = j - ch * PATCH
      band_b[ch, y, pl.ds(0, LANES)] = _tanh16(
          rows8[7, pl.ds(pl.multiple_of(j * LANES, 16), LANES)])

    def fan_col(c, _):
      cbase = pl.multiple_of(c * PATCH, 16)

      @plsc.parallel_loop(0, VECS, unroll=4)
      def _(j):
        ch = j // PATCH
        y = j - ch * PATCH
        band_b[ch, y, pl.ds(cbase, LANES)] = band_b[ch, y, pl.ds(0, LANES)]

      return 0
    lax.fori_loop(1, ROWS, fan_col, 0)

    def write_rep(r, _):
      band_write(n, r, band_b, rsem).start()
      return 0
    lax.fori_loop(9, ROWS, write_rep, 0)

    band_write(n, 8, band_a0, ws0).wait()
    build_cols(rows8, band_a0, 0, 7)

    def fill_199(c, _):
      cbase = pl.multiple_of(c * PATCH, 16)

      @plsc.parallel_loop(0, VECS, unroll=4)
      def _(j):
        ch = j // PATCH
        y = j - ch * PATCH
        band_a0[ch, y, pl.ds(cbase, LANES)] = band_b[ch, y, pl.ds(0, LANES)]

      return 0
    lax.fori_loop(7, ROWS, fill_199, 0)
    band_write(n, 8, band_a0, ws0).start()
    return 0

  lax.fori_loop(0, SPW, do_sample, 0)

  last = wid * SPW + (SPW - 1)
  band_write(last, 8, band_a0, ws0).wait()
  band_write(last, 7, band_a1, ws1).wait()
  for _k in range(ROWS - 9):
    band_write(last, 9, band_b, rsem).wait()


@jax.jit
def kernel(sentence_batch, emb_table):
  mesh = plsc.VectorSubcoreMesh(core_axis_name="c", subcore_axis_name="s",
                                num_cores=NUM_CORES,
                                num_subcores=NUM_SUBCORES)
  run = pl.kernel(
      _sc_body,
      out_type=jax.ShapeDtypeStruct((N, 3, IMG, IMG), jnp.float32),
      mesh=mesh,
      scratch_types=[
          pltpu.VMEM((SPW * L,), jnp.int32),
          pltpu.VMEM((ROWS, EMB_DIM), jnp.float32),
          pltpu.VMEM((ROWS, EMB_DIM), jnp.float32),
          pltpu.VMEM((8, EMB_DIM), jnp.float32),
          pltpu.VMEM((3, PATCH, IMG), jnp.float32),
          pltpu.VMEM((3, PATCH, IMG), jnp.float32),
          pltpu.VMEM((3, PATCH, IMG), jnp.float32),
          pltpu.SemaphoreType.DMA,
          pltpu.SemaphoreType.DMA,
          pltpu.SemaphoreType.DMA,
          pltpu.SemaphoreType.DMA,
          pltpu.SemaphoreType.DMA,
          pltpu.SemaphoreType.DMA,
      ],
  )
  return run(sentence_batch.astype(jnp.int32).reshape(N * L), emb_table)
